# Initial kernel scaffold; baseline (speedup 1.0000x reference)
import jax
import jax.numpy as jnp
from jax.experimental import pallas as pl


def kernel(x, indices, W, b):
    raise NotImplementedError("write your pallas kernel here")


# speedup vs baseline: 2.3634x; 2.3634x over previous
import jax
import jax.numpy as jnp
from jax.experimental import pallas as pl
from jax.experimental.pallas import tpu as pltpu


def _fc_body(order_ref, sidx_ref, x_ref, w_ref, b_ref, o_ref):
    y = jax.lax.dot_general(
        x_ref[0], w_ref[0],
        dimension_numbers=(((1,), (1,)), ((), ())),
        preferred_element_type=jnp.float32,
    )
    o_ref[0] = y + b_ref[0]


def kernel(x, indices, W, b):
    B, S, D_IN = x.shape
    E, D_OUT, _ = W.shape
    idx = indices.astype(jnp.int32)
    order = jnp.argsort(idx).astype(jnp.int32)
    sidx = jnp.take(idx, order)
    b3 = b.reshape(E, 1, D_OUT)

    grid_spec = pltpu.PrefetchScalarGridSpec(
        num_scalar_prefetch=2,
        grid=(B,),
        in_specs=[
            pl.BlockSpec((1, S, D_IN), lambda i, o, s: (o[i], 0, 0)),
            pl.BlockSpec((1, D_OUT, D_IN), lambda i, o, s: (s[i], 0, 0)),
            pl.BlockSpec((1, 1, D_OUT), lambda i, o, s: (s[i], 0, 0)),
        ],
        out_specs=pl.BlockSpec((1, S, D_OUT), lambda i, o, s: (o[i], 0, 0)),
    )
    return pl.pallas_call(
        _fc_body,
        grid_spec=grid_spec,
        out_shape=jax.ShapeDtypeStruct((B, S, D_OUT), jnp.float32),
    )(order, sidx, x, W, b3)


# speedup vs baseline: 2.7748x; 1.1741x over previous
import functools

import jax
import jax.numpy as jnp
from jax.experimental import pallas as pl
from jax.experimental.pallas import tpu as pltpu


def _fc_body(uniq_ref, start_ref, count_ref, order_ref,
             x_ref, w_ref, b_ref, o_ref, xt_ref, *, B, S, T):
    j = pl.program_id(0)
    cnt = count_ref[j]
    st = start_ref[j]

    @pl.when(cnt > 0)
    def _run():
        n_tiles = (cnt + T - 1) // T

        def tile_body(t, carry):
            base = st + t * T
            lim = st + cnt
            for k in range(T):
                pos = jnp.minimum(base + k, B - 1)
                src = order_ref[pos]
                xt_ref[k * S:(k + 1) * S, :] = x_ref[src]
            y = jax.lax.dot_general(
                xt_ref[:, :], w_ref[0],
                dimension_numbers=(((1,), (1,)), ((), ())),
                preferred_element_type=jnp.float32,
            ) + b_ref[0]
            for k in range(T):
                p = base + k

                @pl.when(p < lim)
                def _store(k=k, p=p):
                    dst = order_ref[jnp.minimum(p, B - 1)]
                    o_ref[dst] = y[k * S:(k + 1) * S, :]
            return carry

        jax.lax.fori_loop(0, n_tiles, tile_body, 0)


def kernel(x, indices, W, b):
    B, S, D_IN = x.shape
    E, D_OUT, _ = W.shape
    T = 8

    idx = indices.astype(jnp.int32)
    order = jnp.argsort(idx).astype(jnp.int32)
    sidx = jnp.take(idx, order)
    is_start = jnp.concatenate(
        [jnp.ones((1,), jnp.bool_), sidx[1:] != sidx[:-1]])
    slot = jnp.cumsum(is_start.astype(jnp.int32)) - 1
    uniq_e = jnp.full((E,), sidx[-1], jnp.int32).at[slot].set(sidx)
    count = jnp.zeros((E,), jnp.int32).at[slot].add(1)
    start = jnp.full((E,), B, jnp.int32).at[slot].min(
        jnp.arange(B, dtype=jnp.int32))

    b3 = b.reshape(E, 1, D_OUT)
    grid_spec = pltpu.PrefetchScalarGridSpec(
        num_scalar_prefetch=4,
        grid=(E,),
        in_specs=[
            pl.BlockSpec((B, S, D_IN), lambda j, u, s, c, o: (0, 0, 0)),
            pl.BlockSpec((1, D_OUT, D_IN), lambda j, u, s, c, o: (u[j], 0, 0)),
            pl.BlockSpec((1, 1, D_OUT), lambda j, u, s, c, o: (u[j], 0, 0)),
        ],
        out_specs=pl.BlockSpec((B, S, D_OUT), lambda j, u, s, c, o: (0, 0, 0)),
        scratch_shapes=[pltpu.VMEM((T * S, D_IN), jnp.float32)],
    )
    return pl.pallas_call(
        functools.partial(_fc_body, B=B, S=S, T=T),
        grid_spec=grid_spec,
        out_shape=jax.ShapeDtypeStruct((B, S, D_OUT), jnp.float32),
    )(uniq_e, start, count, order, x, W, b3)


# speedup vs baseline: 2.8366x; 1.0223x over previous
import functools

import jax
import jax.numpy as jnp
from jax.experimental import pallas as pl
from jax.experimental.pallas import tpu as pltpu

_NSLOT = 2
_NCHUNK = 8


def _fc_body(nu_ref, uniq_ref, start_ref, count_ref, order_ref,
             x_ref, w_hbm, b_ref, o_ref, wbuf, xt_ref, sem,
             *, B, S, T, D_OUT, D_IN):
    nu = nu_ref[0]
    rows = D_OUT // _NCHUNK

    def start_fetch(i):
        slot = jax.lax.rem(i, _NSLOT)
        e = uniq_ref[i]
        for c in range(_NCHUNK):
            pltpu.make_async_copy(
                w_hbm.at[e, pl.ds(c * rows, rows), :],
                wbuf.at[slot, pl.ds(c * rows, rows), :],
                sem.at[slot, c],
            ).start()

    def wait_fetch(i):
        slot = jax.lax.rem(i, _NSLOT)
        e = uniq_ref[i]
        for c in range(_NCHUNK):
            pltpu.make_async_copy(
                w_hbm.at[e, pl.ds(c * rows, rows), :],
                wbuf.at[slot, pl.ds(c * rows, rows), :],
                sem.at[slot, c],
            ).wait()

    start_fetch(0)

    def expert_body(i, carry):
        slot = jax.lax.rem(i, _NSLOT)

        @pl.when(i + 1 < nu)
        def _prefetch():
            start_fetch(i + 1)

        wait_fetch(i)

        e = uniq_ref[i]
        cnt = count_ref[i]
        st = start_ref[i]
        n_tiles = (cnt + T - 1) // T

        def tile_body(t, c2):
            base = st + t * T
            lim = st + cnt
            for k in range(T):
                pos = jnp.minimum(base + k, B - 1)
                src = order_ref[pos]
                xt_ref[k * S:(k + 1) * S, :] = x_ref[src]
            y = jax.lax.dot_general(
                xt_ref[:, :], wbuf[slot],
                dimension_numbers=(((1,), (1,)), ((), ())),
                preferred_element_type=jnp.float32,
            ) + b_ref[pl.ds(e, 1), :]
            for k in range(T):
                p = base + k

                @pl.when(p < lim)
                def _store(k=k, p=p):
                    dst = order_ref[jnp.minimum(p, B - 1)]
                    o_ref[dst] = y[k * S:(k + 1) * S, :]
            return c2

        jax.lax.fori_loop(0, n_tiles, tile_body, 0)
        return carry

    jax.lax.fori_loop(0, nu, expert_body, 0)


def kernel(x, indices, W, b):
    B, S, D_IN = x.shape
    E, D_OUT, _ = W.shape
    T = 8

    idx = indices.astype(jnp.int32)
    order = jnp.argsort(idx).astype(jnp.int32)
    sidx = jnp.take(idx, order)
    is_start = jnp.concatenate(
        [jnp.ones((1,), jnp.bool_), sidx[1:] != sidx[:-1]])
    slot = jnp.cumsum(is_start.astype(jnp.int32)) - 1
    nu = slot[-1:] + 1
    uniq_e = jnp.zeros((E,), jnp.int32).at[slot].set(sidx)
    count = jnp.zeros((E,), jnp.int32).at[slot].add(1)
    start = jnp.full((E,), B, jnp.int32).at[slot].min(
        jnp.arange(B, dtype=jnp.int32))

    return pl.pallas_call(
        functools.partial(_fc_body, B=B, S=S, T=T, D_OUT=D_OUT, D_IN=D_IN),
        in_specs=[
            pl.BlockSpec(memory_space=pltpu.SMEM),
            pl.BlockSpec(memory_space=pltpu.SMEM),
            pl.BlockSpec(memory_space=pltpu.SMEM),
            pl.BlockSpec(memory_space=pltpu.SMEM),
            pl.BlockSpec(memory_space=pltpu.SMEM),
            pl.BlockSpec(memory_space=pltpu.VMEM),
            pl.BlockSpec(memory_space=pl.ANY),
            pl.BlockSpec(memory_space=pltpu.VMEM),
        ],
        out_specs=pl.BlockSpec(memory_space=pltpu.VMEM),
        scratch_shapes=[
            pltpu.VMEM((_NSLOT, D_OUT, D_IN), jnp.float32),
            pltpu.VMEM((T * S, D_IN), jnp.float32),
            pltpu.SemaphoreType.DMA((_NSLOT, _NCHUNK)),
        ],
        out_shape=jax.ShapeDtypeStruct((B, S, D_OUT), jnp.float32),
    )(nu, uniq_e, start, count, order, x, W, b)


# speedup vs baseline: 3.2983x; 1.1627x over previous
import functools

import jax
import jax.numpy as jnp
from jax.experimental import pallas as pl
from jax.experimental.pallas import tpu as pltpu

_NSLOT = 3
_NCHUNK = 1


def _fc_body(nu_ref, uniq_ref, start_ref, count_ref, order_ref,
             x_ref, w_hbm, b_ref, o_ref, wbuf, xt_ref, sem,
             *, B, S, T, D_OUT, D_IN):
    nu = nu_ref[0]
    rows = D_OUT // _NCHUNK

    def start_fetch(i):
        slot = jax.lax.rem(i, _NSLOT)
        e = uniq_ref[i]
        for c in range(_NCHUNK):
            pltpu.make_async_copy(
                w_hbm.at[e, pl.ds(c * rows, rows), :],
                wbuf.at[slot, pl.ds(c * rows, rows), :],
                sem.at[slot, c],
            ).start()

    def wait_fetch(i):
        slot = jax.lax.rem(i, _NSLOT)
        e = uniq_ref[i]
        for c in range(_NCHUNK):
            pltpu.make_async_copy(
                w_hbm.at[e, pl.ds(c * rows, rows), :],
                wbuf.at[slot, pl.ds(c * rows, rows), :],
                sem.at[slot, c],
            ).wait()

    start_fetch(0)

    @pl.when(nu > 1)
    def _pro2():
        start_fetch(1)

    def expert_body(i, carry):
        slot = jax.lax.rem(i, _NSLOT)

        @pl.when(i + 2 < nu)
        def _prefetch():
            start_fetch(i + 2)

        wait_fetch(i)

        e = uniq_ref[i]
        cnt = count_ref[i]
        st = start_ref[i]
        n_tiles = (cnt + T - 1) // T

        def tile_body(t, c2):
            base = st + t * T
            lim = st + cnt
            for k in range(T):
                pos = jnp.minimum(base + k, B - 1)
                src = order_ref[pos]
                xt_ref[k * S:(k + 1) * S, :] = x_ref[src]
            y = jax.lax.dot_general(
                xt_ref[:, :], wbuf[slot],
                dimension_numbers=(((1,), (1,)), ((), ())),
                preferred_element_type=jnp.float32,
            ) + b_ref[pl.ds(e, 1), :]
            for k in range(T):
                p = base + k

                @pl.when(p < lim)
                def _store(k=k, p=p):
                    dst = order_ref[jnp.minimum(p, B - 1)]
                    o_ref[dst] = y[k * S:(k + 1) * S, :]
            return c2

        jax.lax.fori_loop(0, n_tiles, tile_body, 0)
        return carry

    jax.lax.fori_loop(0, nu, expert_body, 0)


def kernel(x, indices, W, b):
    B, S, D_IN = x.shape
    E, D_OUT, _ = W.shape
    T = 8

    idx = indices.astype(jnp.int32)
    order = jnp.argsort(idx).astype(jnp.int32)
    sidx = jnp.take(idx, order)
    is_start = jnp.concatenate(
        [jnp.ones((1,), jnp.bool_), sidx[1:] != sidx[:-1]])
    slot = jnp.cumsum(is_start.astype(jnp.int32)) - 1
    nu = slot[-1:] + 1
    uniq_e = jnp.zeros((E,), jnp.int32).at[slot].set(sidx)
    count = jnp.zeros((E,), jnp.int32).at[slot].add(1)
    start = jnp.full((E,), B, jnp.int32).at[slot].min(
        jnp.arange(B, dtype=jnp.int32))

    return pl.pallas_call(
        functools.partial(_fc_body, B=B, S=S, T=T, D_OUT=D_OUT, D_IN=D_IN),
        in_specs=[
            pl.BlockSpec(memory_space=pltpu.SMEM),
            pl.BlockSpec(memory_space=pltpu.SMEM),
            pl.BlockSpec(memory_space=pltpu.SMEM),
            pl.BlockSpec(memory_space=pltpu.SMEM),
            pl.BlockSpec(memory_space=pltpu.SMEM),
            pl.BlockSpec(memory_space=pltpu.VMEM),
            pl.BlockSpec(memory_space=pl.ANY),
            pl.BlockSpec(memory_space=pltpu.VMEM),
        ],
        out_specs=pl.BlockSpec(memory_space=pltpu.VMEM),
        scratch_shapes=[
            pltpu.VMEM((_NSLOT, D_OUT, D_IN), jnp.float32),
            pltpu.VMEM((T * S, D_IN), jnp.float32),
            pltpu.SemaphoreType.DMA((_NSLOT, _NCHUNK)),
        ],
        out_shape=jax.ShapeDtypeStruct((B, S, D_OUT), jnp.float32),
    )(nu, uniq_e, start, count, order, x, W, b)
